# baseline (speedup 1.0000x reference)
import jax
import jax.numpy as jnp
from jax.experimental import pallas as pl


def _mean_heads(x_ref, o_ref):
    o_ref[...] = jnp.mean(x_ref[...], axis=-2)


def _mean_heads_flat(x_ref, o_ref):
    x = x_ref[...]
    o_ref[...] = 0.25 * (x[:, 0:16] + x[:, 16:32] + x[:, 32:48] + x[:, 48:64])


def kernel(nfeats, efeats, edge_index, W_ni, W_nj, W_fij, W_node, attn, bias):
    N = nfeats.shape[0]
    E = efeats.shape[0]
    H = attn.shape[1]
    OUT_E = attn.shape[2]
    OUT_N = W_node.shape[1] // H
    src = edge_index[0]
    dst = edge_index[1]
    f_ni = nfeats @ W_ni
    f_nj = nfeats @ W_nj
    f_fij = efeats @ W_fij
    f_tmp = jnp.take(f_ni, src, axis=0) + jnp.take(f_nj, dst, axis=0)
    f_out = jax.nn.leaky_relu(f_tmp + f_fij + bias)
    f_out = f_out.reshape(E, H, OUT_E)
    e = jnp.sum(f_out * attn, axis=-1)
    e_max = jax.ops.segment_max(e, dst, num_segments=N)
    e_max = jnp.where(jnp.isfinite(e_max), e_max, 0.0)
    e_exp = jnp.exp(e - jnp.take(e_max, dst, axis=0))
    e_sum = jax.ops.segment_sum(e_exp, dst, num_segments=N)
    a = e_exp / jnp.take(e_sum, dst, axis=0)
    h = (nfeats @ W_node).reshape(N, H, OUT_N)
    m = jnp.take(h, src, axis=0) * a[:, :, None]
    h_out = jax.ops.segment_sum(m, dst, num_segments=N)
    rn = h_out.mean(axis=-2)
    re = f_out.mean(axis=-2)
    return rn, re


# speedup vs baseline: 13.2914x; 13.2914x over previous
import dataclasses
import functools

import jax
import jax.numpy as jnp
from jax import lax
from jax.experimental import pallas as pl
from jax.experimental.pallas import tpu as pltpu
from jax.experimental.pallas import tpu_sc as plsc

NEG = -1e30
F32 = jnp.float32

_SC_PARAMS = pltpu.CompilerParams()
if "needs_layout_passes" in pltpu.CompilerParams.__dataclass_fields__:
    _SC_PARAMS = dataclasses.replace(_SC_PARAMS, needs_layout_passes=False)


def _proj_body(x_ref, wcat_ref, fcat_ref):
    fcat_ref[...] = jnp.dot(x_ref[...], wcat_ref[...],
                            preferred_element_type=F32)


def _k2_body(e_pad, ho, fcat_hbm, src_hbm, dst_hbm, fsum_hbm,
             si_v, a_v, b_v):
    c = lax.axis_index("c")
    s = lax.axis_index("s")
    wid = s * 2 + c
    per = e_pad // 32
    base0 = wid * per

    @pl.loop(0, per // 128)
    def _chunk(j):
        base = base0 + j * 128
        pltpu.sync_copy(src_hbm.at[pl.ds(base, 128)], si_v)
        pltpu.sync_copy(fcat_hbm.at[si_v], a_v)
        pltpu.sync_copy(dst_hbm.at[pl.ds(base, 128)], si_v)
        pltpu.sync_copy(fcat_hbm.at[si_v], b_v)

        @pl.loop(0, 128)
        def _row(i):
            ar = a_v.at[i]
            br = b_v.at[i]
            for k in range(ho // 16):
                ar[pl.ds(k * 16, 16)] = (
                    ar[pl.ds(k * 16, 16)] + br[pl.ds(ho + k * 16, 16)])

        pltpu.sync_copy(a_v, fsum_hbm.at[pl.ds(base, 128)])


def _k3_body(be, e_real, fsum_ref, ef_ref, wf_ref, ablk_ref, mmean_ref,
             bias_ref, re_ref, et_ref, c_ref):
    i = pl.program_id(0)
    f = fsum_ref[...][:, :bias_ref.shape[1]] + jnp.dot(
        ef_ref[...], wf_ref[...], preferred_element_type=F32) + bias_ref[...]
    f = jnp.where(f >= 0, f, 0.01 * f)
    re_ref[...] = jnp.dot(f, mmean_ref[...], preferred_element_type=F32)
    et = lax.dot_general(ablk_ref[...], f, (((0,), (1,)), ((), ())),
                         preferred_element_type=F32)
    ids = i * be + lax.broadcasted_iota(jnp.int32, et.shape, 1)
    et = jnp.where(ids < e_real, et, NEG)
    et_ref[...] = et

    @pl.when(i == 0)
    def _():
        c_ref[...] = jnp.full_like(c_ref[...], NEG)

    c_ref[...] = jnp.maximum(c_ref[...], jnp.max(et))


def _k6_body(e_pad, n_nodes, in_n, nf_hbm, src_hbm, dst_hbm, et_hbm, c_hbm,
             g_hbm, s_hbm, si_v, di_v, x_v, e_v, p_v, c_v, z_v, gsp, ssp):
    core = lax.axis_index("c")
    tid = lax.axis_index("s")
    per = e_pad // 16
    nchunks = per // 128
    sp = ssp.shape[0]
    rows_main = (n_nodes // 128) * 8
    rows_extra = n_nodes - 16 * rows_main
    zrows = rows_main // 6
    swords = sp // 16

    pltpu.sync_copy(c_hbm.at[0, pl.ds(0, 16)], c_v)

    @pl.loop(0, swords // 16)
    def _z2(i):
        z_v[pl.ds(i * 16, 16)] = jnp.zeros((16,), F32)

    for k in range(2):
        h = core * 2 + k
        @pl.loop(0, 128)
        def _zx(i):
            xr = x_v.at[i]
            for q in range(in_n // 16):
                xr[pl.ds(q * 16, 16)] = jnp.zeros((16,), F32)

        for z5 in range(6):
            pltpu.sync_copy(
                x_v.at[pl.ds(0, zrows)],
                gsp.at[pl.ds(tid * rows_main + z5 * zrows, zrows)])

        @pl.when(tid == 15)
        def _():
            pltpu.sync_copy(x_v.at[pl.ds(0, rows_extra)],
                            gsp.at[pl.ds(16 * rows_main, rows_extra)])

        pltpu.sync_copy(z_v, ssp.at[pl.ds(tid * swords, swords)])
        plsc.subcore_barrier()

        @pl.loop(0, nchunks)
        def _chunk(j):
            base = tid * per + j * 128
            pltpu.sync_copy(src_hbm.at[pl.ds(base, 128)], si_v)
            pltpu.sync_copy(dst_hbm.at[pl.ds(base, 128)], di_v.at[0])
            pltpu.sync_copy(et_hbm.at[h, pl.ds(base, 128)], e_v)
            pltpu.sync_copy(nf_hbm.at[si_v], x_v)
            cc = c_v[...]
            for q in range(8):
                sl = pl.ds(q * 16, 16)
                p_v[sl] = jnp.exp(e_v[sl] - cc)

            @pl.loop(0, 128)
            def _edge(i):
                ps = plsc.load_gather(p_v, [jnp.full((16,), i, jnp.int32)])
                xr = x_v.at[i]
                for q in range(in_n // 16):
                    sl = pl.ds(q * 16, 16)
                    xr[sl] = xr[sl] * ps

            pltpu.sync_copy(x_v, gsp.at[di_v.at[0]], add=True)
            pltpu.sync_copy(p_v, ssp.at[di_v.at[0]], add=True)

        plsc.subcore_barrier()
        pltpu.sync_copy(
            gsp.at[pl.ds(tid * rows_main, rows_main)],
            g_hbm.at[h, pl.ds(tid * rows_main, rows_main)])

        @pl.when(tid == 15)
        def _():
            pltpu.sync_copy(gsp.at[pl.ds(16 * rows_main, rows_extra)],
                            g_hbm.at[h, pl.ds(16 * rows_main, rows_extra)])

        @pl.when(tid == 0)
        def _():
            pltpu.sync_copy(ssp, s_hbm.at[pl.ds(h * sp, sp)])

        plsc.subcore_barrier()


def _k7_body(heads, out_n, g_ref, s_ref, wn_ref, rn_ref):
    g = g_ref[...]
    s = s_ref[...]
    wn = wn_ref[...]
    acc = jnp.zeros(rn_ref.shape, F32)
    for h in range(heads):
        sh = s[h]
        inv = jnp.where(sh > 0, 1.0 / sh, 0.0)[:, None]
        acc = acc + jnp.dot(g[h] * inv, wn[:, h * out_n:(h + 1) * out_n],
                            preferred_element_type=F32)
    rn_ref[...] = (1.0 / heads) * acc


def kernel(nfeats, efeats, edge_index, W_ni, W_nj, W_fij, W_node, attn, bias):
    N, IN_N = nfeats.shape
    E, IN_E = efeats.shape
    H = attn.shape[1]
    OUT_E = attn.shape[2]
    OUT_N = W_node.shape[1] // H
    HO = H * OUT_E
    E_pad = ((E + 4095) // 4096) * 4096
    pad = E_pad - E

    src_p = jnp.pad(edge_index[0], (0, pad))
    dst_p = jnp.pad(edge_index[1], (0, pad))
    ef_p = jnp.pad(efeats, ((0, pad), (0, 0)))

    Ablk = (attn[0][:, :, None] * jnp.eye(H, dtype=F32)[:, None, :])
    Ablk = Ablk.reshape(HO, H)
    Ablk = jnp.pad(Ablk, ((0, 0), (0, 8 - H)))
    Mmean = jnp.tile(jnp.eye(OUT_E, dtype=F32), (H, 1)) * (1.0 / H)
    bias2 = bias.reshape(1, HO).astype(F32)

    Wcat = jnp.concatenate([W_ni, W_nj], axis=1)
    BN1 = 2000
    fcat = pl.pallas_call(
        _proj_body,
        grid=(N // BN1,),
        in_specs=[
            pl.BlockSpec((BN1, IN_N), lambda i: (i, 0)),
            pl.BlockSpec((IN_N, 2 * HO), lambda i: (0, 0)),
        ],
        out_specs=pl.BlockSpec((BN1, 2 * HO), lambda i: (i, 0)),
        out_shape=jax.ShapeDtypeStruct((N, 2 * HO), F32),
    )(nfeats, Wcat)

    mesh = plsc.VectorSubcoreMesh(core_axis_name="c", subcore_axis_name="s")
    fsum = pl.kernel(
        functools.partial(_k2_body, E_pad, HO),
        out_type=jax.ShapeDtypeStruct((E_pad, 2 * HO), F32),
        mesh=mesh,
        scratch_types=[
            pltpu.VMEM((128,), jnp.int32),
            pltpu.VMEM((128, 2 * HO), F32),
            pltpu.VMEM((128, 2 * HO), F32),
        ],
    )(fcat, src_p, dst_p)

    BE = 2048
    re_p, et, Carr = pl.pallas_call(
        functools.partial(_k3_body, BE, E),
        grid=(E_pad // BE,),
        in_specs=[
            pl.BlockSpec((BE, 2 * HO), lambda i: (i, 0)),
            pl.BlockSpec((BE, IN_E), lambda i: (i, 0)),
            pl.BlockSpec((IN_E, HO), lambda i: (0, 0)),
            pl.BlockSpec((HO, 8), lambda i: (0, 0)),
            pl.BlockSpec((HO, OUT_E), lambda i: (0, 0)),
            pl.BlockSpec((1, HO), lambda i: (0, 0)),
        ],
        out_specs=(
            pl.BlockSpec((BE, OUT_E), lambda i: (i, 0)),
            pl.BlockSpec((8, BE), lambda i: (0, i)),
            pl.BlockSpec((8, 128), lambda i: (0, 0)),
        ),
        out_shape=(
            jax.ShapeDtypeStruct((E_pad, OUT_E), F32),
            jax.ShapeDtypeStruct((8, E_pad), F32),
            jax.ShapeDtypeStruct((8, 128), F32),
        ),
    )(fsum, ef_p, W_fij, Ablk, Mmean, bias2)

    SP = ((N + 639) // 640) * 640
    g, s = pl.kernel(
        functools.partial(_k6_body, E_pad, N, IN_N),
        out_type=(
            jax.ShapeDtypeStruct((H, N, IN_N), F32),
            jax.ShapeDtypeStruct((H * SP,), F32),
        ),
        mesh=plsc.VectorSubcoreMesh(core_axis_name="c", subcore_axis_name="s"),
        compiler_params=_SC_PARAMS,
        scratch_types=[
            pltpu.VMEM((128,), jnp.int32),
            pltpu.VMEM((1, 128), jnp.int32),
            pltpu.VMEM((128, IN_N), F32),
            pltpu.VMEM((128,), F32),
            pltpu.VMEM((128,), F32),
            pltpu.VMEM((16,), F32),
            pltpu.VMEM((SP // 16,), F32),
            pltpu.VMEM_SHARED((N, IN_N), F32),
            pltpu.VMEM_SHARED((SP,), F32),
        ],
    )(nfeats, src_p, dst_p, et, Carr)

    s2 = s.reshape(H, SP)
    BN7 = 2048
    rn = pl.pallas_call(
        functools.partial(_k7_body, H, OUT_N),
        grid=((N + BN7 - 1) // BN7,),
        in_specs=[
            pl.BlockSpec((H, BN7, IN_N), lambda i: (0, i, 0)),
            pl.BlockSpec((H, BN7), lambda i: (0, i)),
            pl.BlockSpec((IN_N, H * OUT_N), lambda i: (0, 0)),
        ],
        out_specs=pl.BlockSpec((BN7, OUT_N), lambda i: (i, 0)),
        out_shape=jax.ShapeDtypeStruct((N, OUT_N), F32),
    )(g, s2, W_node)

    return rn, re_p[:E]
